# Initial kernel scaffold; baseline (speedup 1.0000x reference)
import jax
import jax.numpy as jnp
from jax.experimental import pallas as pl


def kernel(x, edge_index, enc_W, enc_b, gin0_W1, gin0_b1, gin0_W2, gin0_b2, gin1_W1, gin1_b1, gin1_W2, gin1_b2, gate_W, gate_b, dec_W, dec_b):
    raise NotImplementedError("write your pallas kernel here")


# speedup vs baseline: 6.7375x; 6.7375x over previous
import functools

import jax
import jax.numpy as jnp
from jax import lax
from jax.experimental import pallas as pl
from jax.experimental.pallas import tpu as pltpu
from jax.experimental.pallas import tpu_sc as plsc

_NC = 2
_NS = 16
_CH = 80


def _seg_sum_partials(h, src3, dst3, zeros):
    n, d = h.shape
    nw = _NC * _NS
    nch = src3.shape[1]
    rpt = n // _NS

    mesh = plsc.VectorSubcoreMesh(core_axis_name="c", subcore_axis_name="s")

    @functools.partial(
        pl.kernel,
        mesh=mesh,
        compiler_params=pltpu.CompilerParams(use_tc_tiling_on_sc=False),
        out_type=jax.ShapeDtypeStruct((_NC * n, d), jnp.float32),
        scratch_types=[
            pltpu.VMEM((nch, _CH), jnp.int32),
            pltpu.VMEM((nch, _CH), jnp.int32),
            pltpu.VMEM((_CH, d), jnp.float32),
            pltpu.VMEM_SHARED((n, d), jnp.float32),
            pltpu.SemaphoreType.DMA,
        ],
    )
    def body(h_hbm, src_hbm, dst_hbm, z_hbm, out_hbm, sidx, didx, rows, acc, sem):
        cid = lax.axis_index("c")
        sid = lax.axis_index("s")
        wid = sid * _NC + cid
        pltpu.sync_copy(z_hbm.at[pl.ds(sid * rpt, rpt)],
                        acc.at[pl.ds(sid * rpt, rpt)])
        pltpu.sync_copy(src_hbm.at[wid], sidx)
        pltpu.sync_copy(dst_hbm.at[wid], didx)
        plsc.subcore_barrier()

        def step(j, carry):
            pltpu.async_copy(h_hbm.at[sidx.at[j]], rows, sem).wait()
            pltpu.sync_copy(rows, acc.at[didx.at[j]], add=True)
            return carry

        lax.fori_loop(0, nch, step, 0)
        plsc.subcore_barrier()
        pltpu.sync_copy(acc.at[pl.ds(sid * rpt, rpt)],
                        out_hbm.at[pl.ds(cid * n + sid * rpt, rpt)])

    return body(h, src3, dst3, zeros)


_DOT = dict(preferred_element_type=jnp.float32, precision=lax.Precision.HIGHEST)


def _enc(x, w, b):
    n, d_in = x.shape
    d_h = w.shape[1]
    br = 1000

    def body(x_ref, w_ref, b_ref, o_ref):
        o_ref[...] = jnp.maximum(
            jnp.dot(x_ref[...], w_ref[...], **_DOT) + b_ref[...], 0.0)

    return pl.pallas_call(
        body,
        grid=(n // br,),
        in_specs=[
            pl.BlockSpec((br, d_in), lambda i: (i, 0)),
            pl.BlockSpec((d_in, d_h), lambda i: (0, 0)),
            pl.BlockSpec((1, d_h), lambda i: (0, 0)),
        ],
        out_specs=pl.BlockSpec((br, d_h), lambda i: (i, 0)),
        out_shape=jax.ShapeDtypeStruct((n, d_h), jnp.float32),
    )(x, w, b.reshape(1, d_h))


def _mlp_gate(p, h, s, w1, b1, w2, b2, gw, gb, dec_w=None, dec_b=None):
    n, d = h.shape
    br = 1000
    final = dec_w is not None
    d_out = dec_w.shape[1] if final else d

    def body(p_ref, h_ref, s_ref, w1_ref, b1_ref, w2_ref, b2_ref, gw_ref,
             gb_ref, *rest):
        z = p_ref[0] + p_ref[1] + h_ref[...]
        t = jnp.maximum(jnp.dot(z, w1_ref[...], **_DOT) + b1_ref[...], 0.0)
        conv = jnp.dot(t, w2_ref[...], **_DOT) + b2_ref[...]
        gl = (jnp.dot(s_ref[...], gw_ref[:d], **_DOT)
              + jnp.dot(conv, gw_ref[d:], **_DOT) + gb_ref[...])
        a = 1.0 / (1.0 + jnp.exp(-gl))
        ns = a * s_ref[...] + (1.0 - a) * conv
        if final:
            dw_ref, db_ref, o_ref = rest
            o_ref[...] = jnp.dot(ns, dw_ref[...], **_DOT) + db_ref[...]
        else:
            o1_ref, o2_ref = rest
            o1_ref[...] = ns
            o2_ref[...] = conv

    in_specs = [
        pl.BlockSpec((2, br, d), lambda i: (0, i, 0)),
        pl.BlockSpec((br, d), lambda i: (i, 0)),
        pl.BlockSpec((br, d), lambda i: (i, 0)),
        pl.BlockSpec((d, d), lambda i: (0, 0)),
        pl.BlockSpec((1, d), lambda i: (0, 0)),
        pl.BlockSpec((d, d), lambda i: (0, 0)),
        pl.BlockSpec((1, d), lambda i: (0, 0)),
        pl.BlockSpec((2 * d, 1), lambda i: (0, 0)),
        pl.BlockSpec((1, 1), lambda i: (0, 0)),
    ]
    args = [p, h, s, w1, b1.reshape(1, d), w2, b2.reshape(1, d), gw,
            gb.reshape(1, 1)]
    if final:
        in_specs += [
            pl.BlockSpec((d, d_out), lambda i: (0, 0)),
            pl.BlockSpec((1, d_out), lambda i: (0, 0)),
        ]
        args += [dec_w, dec_b.reshape(1, d_out)]
        out_specs = pl.BlockSpec((br, d_out), lambda i: (i, 0))
        out_shape = jax.ShapeDtypeStruct((n, d_out), jnp.float32)
    else:
        out_specs = [
            pl.BlockSpec((br, d), lambda i: (i, 0)),
            pl.BlockSpec((br, d), lambda i: (i, 0)),
        ]
        out_shape = [
            jax.ShapeDtypeStruct((n, d), jnp.float32),
            jax.ShapeDtypeStruct((n, d), jnp.float32),
        ]

    return pl.pallas_call(
        body,
        grid=(n // br,),
        in_specs=in_specs,
        out_specs=out_specs,
        out_shape=out_shape,
    )(*args)


def kernel(x, edge_index, enc_W, enc_b, gin0_W1, gin0_b1, gin0_W2, gin0_b2,
           gin1_W1, gin1_b1, gin1_W2, gin1_b2, gate_W, gate_b, dec_W, dec_b):
    n = x.shape[0]
    d = enc_W.shape[1]
    e = edge_index.shape[1]
    nw = _NC * _NS
    nch = e // (nw * _CH)
    src3 = edge_index[0].reshape(nw, nch, _CH)
    dst3 = edge_index[1].reshape(nw, nch, _CH)
    zeros = jnp.zeros((n, d), jnp.float32)

    init_x = _enc(x, enc_W, enc_b)

    p0 = _seg_sum_partials(init_x, src3, dst3, zeros).reshape(2, n, d)
    self_x, conv_x = _mlp_gate(p0, init_x, init_x, gin0_W1, gin0_b1,
                               gin0_W2, gin0_b2, gate_W, gate_b)
    p1 = _seg_sum_partials(conv_x, src3, dst3, zeros).reshape(2, n, d)
    return _mlp_gate(p1, conv_x, self_x, gin1_W1, gin1_b1, gin1_W2, gin1_b2,
                     gate_W, gate_b, dec_W, dec_b)
